# baseline (speedup 1.0000x reference)
import jax
import jax.numpy as jnp
from jax import lax
from jax.experimental import pallas as pl
from jax.experimental.pallas import tpu as pltpu
from jax.experimental.pallas import tpu_sc as plsc

N = 10000
E = 320000
D = 128
SLOPE = 0.1

NC, NS, L = 2, 16, 16
NW = NC * NS
EPW = E // NW
K = 80
NCHUNK = EPW // K
NRCHUNK = N // K
UNROLL = 6
NLOOP = 120


def _prep_body(f_ref, w_ref, b_ref, a_ref, emb_ref, al_ref):
    emb = jnp.dot(f_ref[...], w_ref[...], preferred_element_type=jnp.float32)
    emb = emb + b_ref[...]
    al_ref[...] = jnp.dot(emb, a_ref[...], preferred_element_type=jnp.float32)
    emb_ref[...] = emb


_prep = pl.pallas_call(
    _prep_body,
    out_shape=[
        jax.ShapeDtypeStruct((N, D), jnp.float32),
        jax.ShapeDtypeStruct((N, 2), jnp.float32),
    ],
)


def _agg_body(emb_hbm, ar_hbm, ac_hbm, r_hbm, c_hbm, num_hbm, den_hbm,
              rbuf, cbuf, abuf, bbuf, ebuf, rows0, rows1, rows2,
              stage0, stage1, acc, dacc,
              gsem0, gsem1, gsem2, ssem0, ssem1,
              isem0, isem1, isem2, isem3, isem4, isem5,
              asem0, asem1, esem0, esem1):
    cid = lax.axis_index("c")
    sid = lax.axis_index("s")
    wid = sid * NC + cid
    rows = (rows0, rows1, rows2)
    stage = (stage0, stage1)
    gsem = (gsem0, gsem1, gsem2)
    ssem = (ssem0, ssem1)
    isem = (isem0, isem1, isem2, isem3, isem4, isem5)
    asem = (asem0, asem1)
    esem = (esem0, esem1)

    z16 = jnp.zeros((L,), jnp.float32)

    def zrow(i, carry):
        for ci in range(D // L):
            stage0[i, pl.ds(ci * L, L)] = z16
        return carry

    lax.fori_loop(0, K, zrow, 0)
    for z in range(NRCHUNK // NS + 1):
        c = z * NS + sid

        @pl.when(c < NRCHUNK)
        def _():
            pltpu.sync_copy(stage0, acc.at[pl.ds(c * K, K)])
            pltpu.sync_copy(stage0.at[0, pl.ds(0, K)], dacc.at[pl.ds(c * K, K)])

    plsc.subcore_barrier()

    def idx_issue(j, q):
        pltpu.async_copy(r_hbm.at[wid, j], rbuf.at[q], isem[q])
        pltpu.async_copy(c_hbm.at[wid, j], cbuf.at[q], isem[q])

    def idx_wait(j, q):
        pltpu.make_async_copy(r_hbm.at[wid, j], rbuf.at[q], isem[q]).wait()
        pltpu.make_async_copy(c_hbm.at[wid, j], cbuf.at[q], isem[q]).wait()

    def ar_issue(j, q, a):
        pltpu.async_copy(ar_hbm.at[rbuf.at[q]], abuf.at[a], asem[a])
        pltpu.async_copy(ac_hbm.at[cbuf.at[q]], bbuf.at[a], asem[a])

    def ar_wait(j, q, a):
        pltpu.make_async_copy(ar_hbm.at[rbuf.at[q]], abuf.at[a], asem[a]).wait()
        pltpu.make_async_copy(ac_hbm.at[cbuf.at[q]], bbuf.at[a], asem[a]).wait()

    def gather(j, q, p):
        pltpu.async_copy(emb_hbm.at[cbuf.at[q]], rows[p], gsem[p])

    def gwait(j, q, p):
        pltpu.make_async_copy(emb_hbm.at[cbuf.at[q]], rows[p], gsem[p]).wait()

    def scatter(j, q, p, a):
        pltpu.async_copy(stage[a], acc.at[rbuf.at[q]], ssem[a], add=True)
        pltpu.async_copy(ebuf.at[a], dacc.at[rbuf.at[q]], esem[a], add=True)

    def swait(j, q, p, a):
        pltpu.make_async_copy(stage[a], acc.at[rbuf.at[q]], ssem[a]).wait()
        pltpu.make_async_copy(ebuf.at[a], dacc.at[rbuf.at[q]], esem[a]).wait()

    lane2 = lax.iota(jnp.int32, L) * 2
    mhi = jnp.full((L,), -65536, jnp.int32)

    def scale(j, a, p):
        rp = rows[p]
        st = stage[a]

        def grp(g, carry):
            s = abuf[a, pl.ds(g * L, L)] + bbuf[a, pl.ds(g * L, L)]
            e16 = jnp.exp(jnp.maximum(s, s * SLOPE))
            ebuf[a, pl.ds(g * L, L)] = e16
            for t in range(L):
                ek = e16[t]
                k = g * L + t
                krow = jnp.full((L,), 0, jnp.int32) + k
                for c0 in range(D // (2 * L)):
                    w = rp[k, pl.ds(c0 * L, L)]
                    lo = plsc.bitcast(w << 16, jnp.float32) * ek
                    hi = plsc.bitcast(w & mhi, jnp.float32) * ek
                    cols = 2 * L * c0 + lane2
                    plsc.store_scatter(st, [krow, cols], lo)
                    plsc.store_scatter(st, [krow, cols + 1], hi)
            return carry

        lax.fori_loop(0, K // L, grp, 0)

    for jj in range(5):
        idx_issue(jj, jj)
    idx_wait(0, 0)
    ar_issue(0, 0, 0)
    gather(0, 0, 0)
    idx_wait(1, 1)
    gather(1, 1, 1)

    def phase(j, m, guard_swait, has_p1, has_p2, has_p5):
        p, q, a = m % 3, m % 6, m % 2
        if has_p1:
            ar_issue(j + 1, (q + 1) % 6, (a + 1) % 2)
        gwait(j, q, p)
        ar_wait(j, q, a)
        scale(j, a, p)
        scatter(j, q, p, a)

        def _wait_prev():
            swait(j - 1, (q + 5) % 6, (p + 2) % 3, (a + 1) % 2)

        if guard_swait:
            pl.when(j > 0)(_wait_prev)
        else:
            _wait_prev()
        if has_p2:
            idx_wait(j + 2, (q + 2) % 6)
            gather(j + 2, (q + 2) % 6, (p + 2) % 3)
        if has_p5:
            idx_issue(j + 5, (q + 5) % 6)

    def sixpack(t, carry):
        j0 = t * UNROLL
        for m in range(UNROLL):
            phase(j0 + m, m, m == 0, True, True, True)
        return carry

    lax.fori_loop(0, NLOOP // UNROLL, sixpack, 0)

    for j in range(NLOOP, NCHUNK):
        m = j % UNROLL
        phase(j, m, False, j + 1 < NCHUNK, j + 2 < NCHUNK, j + 5 < NCHUNK)
    swait(NCHUNK - 1, (NCHUNK - 1) % 6, (NCHUNK - 1) % 3, (NCHUNK - 1) % 2)

    plsc.subcore_barrier()
    for z in range(NRCHUNK // NS + 1):
        c = z * NS + sid

        @pl.when(c < NRCHUNK)
        def _():
            pltpu.sync_copy(acc.at[pl.ds(c * K, K)],
                            num_hbm.at[cid, pl.ds(c * K, K)])
            pltpu.sync_copy(dacc.at[pl.ds(c * K, K)],
                            den_hbm.at[cid, pl.ds(c * K, K)])


_agg = pl.kernel(
    _agg_body,
    out_type=[
        jax.ShapeDtypeStruct((NC, N, D), jnp.float32),
        jax.ShapeDtypeStruct((NC, N), jnp.float32),
    ],
    mesh=plsc.VectorSubcoreMesh(core_axis_name="c", subcore_axis_name="s"),
    compiler_params=pltpu.CompilerParams(
        needs_layout_passes=False, use_tc_tiling_on_sc=False),
    scratch_types=[
        pltpu.VMEM((6, K), jnp.int32),
        pltpu.VMEM((6, K), jnp.int32),
        pltpu.VMEM((2, K), jnp.float32),
        pltpu.VMEM((2, K), jnp.float32),
        pltpu.VMEM((2, K), jnp.float32),
        pltpu.VMEM((K, D // 2), jnp.int32),
        pltpu.VMEM((K, D // 2), jnp.int32),
        pltpu.VMEM((K, D // 2), jnp.int32),
        pltpu.VMEM((K, D), jnp.float32),
        pltpu.VMEM((K, D), jnp.float32),
        pltpu.VMEM_SHARED((N, D), jnp.float32),
        pltpu.VMEM_SHARED((N,), jnp.float32),
        pltpu.SemaphoreType.DMA,
        pltpu.SemaphoreType.DMA,
        pltpu.SemaphoreType.DMA,
        pltpu.SemaphoreType.DMA,
        pltpu.SemaphoreType.DMA,
        pltpu.SemaphoreType.DMA,
        pltpu.SemaphoreType.DMA,
        pltpu.SemaphoreType.DMA,
        pltpu.SemaphoreType.DMA,
        pltpu.SemaphoreType.DMA,
        pltpu.SemaphoreType.DMA,
        pltpu.SemaphoreType.DMA,
        pltpu.SemaphoreType.DMA,
        pltpu.SemaphoreType.DMA,
        pltpu.SemaphoreType.DMA,
    ],
)


def _comb_body(p_ref, d_ref, o_ref):
    num = p_ref[0] + p_ref[1]
    den = d_ref[0] + d_ref[1]
    o_ref[...] = num / (den + 1e-8)


_comb = pl.pallas_call(
    _comb_body,
    out_shape=jax.ShapeDtypeStruct((N, D), jnp.float32),
)


def kernel(features, W, b, a, edge_index, nodes, ind):
    a2d = jnp.concatenate([a[:D], a[D:]], axis=1)
    emb, alphas = _prep(features, W, b.reshape(1, D), a2d)
    packed = lax.bitcast_convert_type(
        emb.astype(jnp.bfloat16).reshape(N, D // 2, 2), jnp.int32)
    r3 = edge_index[0].reshape(NW, NCHUNK, K)
    c3 = edge_index[1].reshape(NW, NCHUNK, K)
    num, den = _agg(packed, alphas[:, 0], alphas[:, 1], r3, c3)
    return _comb(num, den.reshape(NC, N, 1))


# speedup vs baseline: 1.1149x; 1.1149x over previous
import jax
import jax.numpy as jnp
from jax import lax
from jax.experimental import pallas as pl
from jax.experimental.pallas import tpu as pltpu
from jax.experimental.pallas import tpu_sc as plsc

N = 10000
E = 320000
D = 128
SLOPE = 0.1

NC, NS, L = 2, 16, 16
NW = NC * NS
EPW = E // NW
K = 80
NCHUNK = EPW // K
NRCHUNK = N // K
UNROLL = 6
NLOOP = 120


def _prep_body(f_ref, w_ref, b_ref, a_ref, emb_ref, al_ref):
    emb = jnp.dot(f_ref[...], w_ref[...], preferred_element_type=jnp.float32)
    emb = emb + b_ref[...]
    al_ref[...] = jnp.dot(emb, a_ref[...], preferred_element_type=jnp.float32)
    emb_ref[...] = emb


_prep = pl.pallas_call(
    _prep_body,
    out_shape=[
        jax.ShapeDtypeStruct((N, D), jnp.float32),
        jax.ShapeDtypeStruct((N, 2), jnp.float32),
    ],
)


def _agg_body(emb_hbm, ar_hbm, ac_hbm, r_hbm, c_hbm, num_hbm, den_hbm,
              rbuf, cbuf, abuf, bbuf, ebuf, rows0, rows1, rows2,
              stage0, stage1, acc, dacc,
              gsem0, gsem1, gsem2, ssem0, ssem1,
              isem0, isem1, isem2, isem3, isem4, isem5,
              asem0, asem1, esem0, esem1):
    cid = lax.axis_index("c")
    sid = lax.axis_index("s")
    wid = sid * NC + cid
    rows = (rows0, rows1, rows2)
    stage = (stage0, stage1)
    gsem = (gsem0, gsem1, gsem2)
    ssem = (ssem0, ssem1)
    isem = (isem0, isem1, isem2, isem3, isem4, isem5)
    asem = (asem0, asem1)
    esem = (esem0, esem1)

    z16 = jnp.zeros((L,), jnp.float32)

    def zrow(i, carry):
        for ci in range(D // L):
            stage0[i, pl.ds(ci * L, L)] = z16
        return carry

    lax.fori_loop(0, K, zrow, 0)
    for z in range(NRCHUNK // NS + 1):
        c = z * NS + sid

        @pl.when(c < NRCHUNK)
        def _():
            pltpu.sync_copy(stage0, acc.at[pl.ds(c * K, K)])
            pltpu.sync_copy(stage0.at[0, pl.ds(0, K)], dacc.at[pl.ds(c * K, K)])

    plsc.subcore_barrier()

    def idx_issue(j, q):
        pltpu.async_copy(r_hbm.at[wid, j], rbuf.at[q], isem[q])
        pltpu.async_copy(c_hbm.at[wid, j], cbuf.at[q], isem[q])

    def idx_wait(j, q):
        pltpu.make_async_copy(r_hbm.at[wid, j], rbuf.at[q], isem[q]).wait()
        pltpu.make_async_copy(c_hbm.at[wid, j], cbuf.at[q], isem[q]).wait()

    def ar_issue(j, q, a):
        pltpu.async_copy(ar_hbm.at[rbuf.at[q]], abuf.at[a], asem[a])
        pltpu.async_copy(ac_hbm.at[cbuf.at[q]], bbuf.at[a], asem[a])

    def ar_wait(j, q, a):
        pltpu.make_async_copy(ar_hbm.at[rbuf.at[q]], abuf.at[a], asem[a]).wait()
        pltpu.make_async_copy(ac_hbm.at[cbuf.at[q]], bbuf.at[a], asem[a]).wait()

    def gather(j, q, p):
        pltpu.async_copy(emb_hbm.at[cbuf.at[q]], rows[p], gsem[p])

    def gwait(j, q, p):
        pltpu.make_async_copy(emb_hbm.at[cbuf.at[q]], rows[p], gsem[p]).wait()

    def scatter(j, q, p, a):
        pltpu.async_copy(stage[a], acc.at[rbuf.at[q]], ssem[a], add=True)
        pltpu.async_copy(ebuf.at[a], dacc.at[rbuf.at[q]], esem[a], add=True)

    def swait(j, q, p, a):
        pltpu.make_async_copy(stage[a], acc.at[rbuf.at[q]], ssem[a]).wait()
        pltpu.make_async_copy(ebuf.at[a], dacc.at[rbuf.at[q]], esem[a]).wait()

    mhi = jnp.full((L,), -65536, jnp.int32)

    def scale(j, a, p):
        rp = rows[p]
        st = stage[a]

        def grp(g, carry):
            s = abuf[a, pl.ds(g * L, L)] + bbuf[a, pl.ds(g * L, L)]
            e16 = jnp.exp(jnp.maximum(s, s * SLOPE))
            ebuf[a, pl.ds(g * L, L)] = e16
            for t in range(L):
                ek = e16[t]
                k = g * L + t
                for c0 in range(D // (2 * L)):
                    w = rp[k, pl.ds(c0 * L, L)]
                    lo = plsc.bitcast(w << 16, jnp.float32) * ek
                    hi = plsc.bitcast(w & mhi, jnp.float32) * ek
                    st[k, pl.ds(c0 * L, L)] = lo
                    st[k, pl.ds(D // 2 + c0 * L, L)] = hi
            return carry

        lax.fori_loop(0, K // L, grp, 0)

    for jj in range(5):
        idx_issue(jj, jj)
    idx_wait(0, 0)
    ar_issue(0, 0, 0)
    gather(0, 0, 0)
    idx_wait(1, 1)
    gather(1, 1, 1)

    def phase(j, m, guard_swait, has_p1, has_p2, has_p5):
        p, q, a = m % 3, m % 6, m % 2
        if has_p1:
            ar_issue(j + 1, (q + 1) % 6, (a + 1) % 2)
        gwait(j, q, p)
        ar_wait(j, q, a)
        scale(j, a, p)
        scatter(j, q, p, a)

        def _wait_prev():
            swait(j - 1, (q + 5) % 6, (p + 2) % 3, (a + 1) % 2)

        if guard_swait:
            pl.when(j > 0)(_wait_prev)
        else:
            _wait_prev()
        if has_p2:
            idx_wait(j + 2, (q + 2) % 6)
            gather(j + 2, (q + 2) % 6, (p + 2) % 3)
        if has_p5:
            idx_issue(j + 5, (q + 5) % 6)

    def sixpack(t, carry):
        j0 = t * UNROLL
        for m in range(UNROLL):
            phase(j0 + m, m, m == 0, True, True, True)
        return carry

    lax.fori_loop(0, NLOOP // UNROLL, sixpack, 0)

    for j in range(NLOOP, NCHUNK):
        m = j % UNROLL
        phase(j, m, False, j + 1 < NCHUNK, j + 2 < NCHUNK, j + 5 < NCHUNK)
    swait(NCHUNK - 1, (NCHUNK - 1) % 6, (NCHUNK - 1) % 3, (NCHUNK - 1) % 2)

    plsc.subcore_barrier()
    for z in range(NRCHUNK // NS + 1):
        c = z * NS + sid

        @pl.when(c < NRCHUNK)
        def _():
            pltpu.sync_copy(acc.at[pl.ds(c * K, K)],
                            num_hbm.at[cid, pl.ds(c * K, K)])
            pltpu.sync_copy(dacc.at[pl.ds(c * K, K)],
                            den_hbm.at[cid, pl.ds(c * K, K)])


_agg = pl.kernel(
    _agg_body,
    out_type=[
        jax.ShapeDtypeStruct((NC, N, D), jnp.float32),
        jax.ShapeDtypeStruct((NC, N), jnp.float32),
    ],
    mesh=plsc.VectorSubcoreMesh(core_axis_name="c", subcore_axis_name="s"),
    compiler_params=pltpu.CompilerParams(
        needs_layout_passes=False, use_tc_tiling_on_sc=False),
    scratch_types=[
        pltpu.VMEM((6, K), jnp.int32),
        pltpu.VMEM((6, K), jnp.int32),
        pltpu.VMEM((2, K), jnp.float32),
        pltpu.VMEM((2, K), jnp.float32),
        pltpu.VMEM((2, K), jnp.float32),
        pltpu.VMEM((K, D // 2), jnp.int32),
        pltpu.VMEM((K, D // 2), jnp.int32),
        pltpu.VMEM((K, D // 2), jnp.int32),
        pltpu.VMEM((K, D), jnp.float32),
        pltpu.VMEM((K, D), jnp.float32),
        pltpu.VMEM_SHARED((N, D), jnp.float32),
        pltpu.VMEM_SHARED((N,), jnp.float32),
        pltpu.SemaphoreType.DMA,
        pltpu.SemaphoreType.DMA,
        pltpu.SemaphoreType.DMA,
        pltpu.SemaphoreType.DMA,
        pltpu.SemaphoreType.DMA,
        pltpu.SemaphoreType.DMA,
        pltpu.SemaphoreType.DMA,
        pltpu.SemaphoreType.DMA,
        pltpu.SemaphoreType.DMA,
        pltpu.SemaphoreType.DMA,
        pltpu.SemaphoreType.DMA,
        pltpu.SemaphoreType.DMA,
        pltpu.SemaphoreType.DMA,
        pltpu.SemaphoreType.DMA,
        pltpu.SemaphoreType.DMA,
    ],
)


def _comb_body(p_ref, d_ref, o_ref):
    num = p_ref[0] + p_ref[1]
    den = d_ref[0] + d_ref[1]
    o_ref[...] = num / (den + 1e-8)


_comb = pl.pallas_call(
    _comb_body,
    out_shape=jax.ShapeDtypeStruct((N, D), jnp.float32),
)


def kernel(features, W, b, a, edge_index, nodes, ind):
    a2d = jnp.concatenate([a[:D], a[D:]], axis=1)
    emb, alphas = _prep(features, W, b.reshape(1, D), a2d)
    embh = emb.astype(jnp.bfloat16)
    packed = lax.bitcast_convert_type(
        jnp.stack([embh[:, :D // 2], embh[:, D // 2:]], axis=-1), jnp.int32)
    r3 = edge_index[0].reshape(NW, NCHUNK, K)
    c3 = edge_index[1].reshape(NW, NCHUNK, K)
    num, den = _agg(packed, alphas[:, 0], alphas[:, 1], r3, c3)
    return _comb(num, den.reshape(NC, N, 1))


# speedup vs baseline: 1.1308x; 1.0142x over previous
import jax
import jax.numpy as jnp
from jax import lax
from jax.experimental import pallas as pl
from jax.experimental.pallas import tpu as pltpu
from jax.experimental.pallas import tpu_sc as plsc

N = 10000
E = 320000
D = 128
SLOPE = 0.1

NC, NS, L = 2, 16, 16
NW = NC * NS
EPW = E // NW
K = 80
NCHUNK = EPW // K
NRCHUNK = N // K
UNROLL = 6
NLOOP = 120


def _rne16(u):
    return u + 0x7FFF + ((u >> 16) & 1)


def _prep_body(f_ref, w_ref, b_ref, a_ref, emb_ref, al_ref):
    emb = jnp.dot(f_ref[...], w_ref[...], preferred_element_type=jnp.float32)
    emb = emb + b_ref[...]
    al_ref[...] = jnp.dot(emb, a_ref[...], preferred_element_type=jnp.float32)
    u = lax.bitcast_convert_type(emb, jnp.int32)
    lo = (_rne16(u[:, :D // 2]) >> 16) & 0xFFFF
    hi = _rne16(u[:, D // 2:]) & -65536
    emb_ref[...] = lo | hi


_prep = pl.pallas_call(
    _prep_body,
    out_shape=[
        jax.ShapeDtypeStruct((N, D // 2), jnp.int32),
        jax.ShapeDtypeStruct((N, 2), jnp.float32),
    ],
)


def _agg_body(emb_hbm, ar_hbm, ac_hbm, r_hbm, c_hbm, num_hbm, den_hbm,
              rbuf, cbuf, abuf, bbuf, ebuf, rows0, rows1, rows2,
              stage0, stage1, acc, dacc,
              gsem0, gsem1, gsem2, ssem0, ssem1,
              isem0, isem1, isem2, isem3, isem4, isem5,
              asem0, asem1, esem0, esem1):
    cid = lax.axis_index("c")
    sid = lax.axis_index("s")
    wid = sid * NC + cid
    rows = (rows0, rows1, rows2)
    stage = (stage0, stage1)
    gsem = (gsem0, gsem1, gsem2)
    ssem = (ssem0, ssem1)
    isem = (isem0, isem1, isem2, isem3, isem4, isem5)
    asem = (asem0, asem1)
    esem = (esem0, esem1)

    z16 = jnp.zeros((L,), jnp.float32)

    def zrow(i, carry):
        for ci in range(D // L):
            stage0[i, pl.ds(ci * L, L)] = z16
        return carry

    lax.fori_loop(0, K, zrow, 0)
    for z in range(NRCHUNK // NS + 1):
        c = z * NS + sid

        @pl.when(c < NRCHUNK)
        def _():
            pltpu.sync_copy(stage0, acc.at[pl.ds(c * K, K)])
            pltpu.sync_copy(stage0.at[0, pl.ds(0, K)], dacc.at[pl.ds(c * K, K)])

    plsc.subcore_barrier()

    def idx_issue(j, q):
        pltpu.async_copy(r_hbm.at[wid, j], rbuf.at[q], isem[q])
        pltpu.async_copy(c_hbm.at[wid, j], cbuf.at[q], isem[q])

    def idx_wait(j, q):
        pltpu.make_async_copy(r_hbm.at[wid, j], rbuf.at[q], isem[q]).wait()
        pltpu.make_async_copy(c_hbm.at[wid, j], cbuf.at[q], isem[q]).wait()

    def ar_issue(j, q, a):
        pltpu.async_copy(ar_hbm.at[rbuf.at[q]], abuf.at[a], asem[a])
        pltpu.async_copy(ac_hbm.at[cbuf.at[q]], bbuf.at[a], asem[a])

    def ar_wait(j, q, a):
        pltpu.make_async_copy(ar_hbm.at[rbuf.at[q]], abuf.at[a], asem[a]).wait()
        pltpu.make_async_copy(ac_hbm.at[cbuf.at[q]], bbuf.at[a], asem[a]).wait()

    def gather(j, q, p):
        pltpu.async_copy(emb_hbm.at[cbuf.at[q]], rows[p], gsem[p])

    def gwait(j, q, p):
        pltpu.make_async_copy(emb_hbm.at[cbuf.at[q]], rows[p], gsem[p]).wait()

    def scatter(j, q, p, a):
        pltpu.async_copy(stage[a], acc.at[rbuf.at[q]], ssem[a], add=True)
        pltpu.async_copy(ebuf.at[a], dacc.at[rbuf.at[q]], esem[a], add=True)

    def swait(j, q, p, a):
        pltpu.make_async_copy(stage[a], acc.at[rbuf.at[q]], ssem[a]).wait()
        pltpu.make_async_copy(ebuf.at[a], dacc.at[rbuf.at[q]], esem[a]).wait()

    mhi = jnp.full((L,), -65536, jnp.int32)

    def scale(j, a, p):
        rp = rows[p]
        st = stage[a]

        def grp(g, carry):
            s = abuf[a, pl.ds(g * L, L)] + bbuf[a, pl.ds(g * L, L)]
            e16 = jnp.exp(jnp.maximum(s, s * SLOPE))
            ebuf[a, pl.ds(g * L, L)] = e16
            for t in range(L):
                ek = e16[t]
                k = g * L + t
                for c0 in range(D // (2 * L)):
                    w = rp[k, pl.ds(c0 * L, L)]
                    lo = plsc.bitcast(w << 16, jnp.float32) * ek
                    hi = plsc.bitcast(w & mhi, jnp.float32) * ek
                    st[k, pl.ds(c0 * L, L)] = lo
                    st[k, pl.ds(D // 2 + c0 * L, L)] = hi
            return carry

        lax.fori_loop(0, K // L, grp, 0)

    for jj in range(5):
        idx_issue(jj, jj)
    idx_wait(0, 0)
    ar_issue(0, 0, 0)
    gather(0, 0, 0)
    idx_wait(1, 1)
    gather(1, 1, 1)

    def phase(j, m, guard_swait, has_p1, has_p2, has_p5):
        p, q, a = m % 3, m % 6, m % 2
        if has_p1:
            ar_issue(j + 1, (q + 1) % 6, (a + 1) % 2)
        gwait(j, q, p)
        ar_wait(j, q, a)
        scale(j, a, p)
        scatter(j, q, p, a)

        def _wait_prev():
            swait(j - 1, (q + 5) % 6, (p + 2) % 3, (a + 1) % 2)

        if guard_swait:
            pl.when(j > 0)(_wait_prev)
        else:
            _wait_prev()
        if has_p2:
            idx_wait(j + 2, (q + 2) % 6)
            gather(j + 2, (q + 2) % 6, (p + 2) % 3)
        if has_p5:
            idx_issue(j + 5, (q + 5) % 6)

    def sixpack(t, carry):
        j0 = t * UNROLL
        for m in range(UNROLL):
            phase(j0 + m, m, m == 0, True, True, True)
        return carry

    lax.fori_loop(0, NLOOP // UNROLL, sixpack, 0)

    for j in range(NLOOP, NCHUNK):
        m = j % UNROLL
        phase(j, m, False, j + 1 < NCHUNK, j + 2 < NCHUNK, j + 5 < NCHUNK)
    swait(NCHUNK - 1, (NCHUNK - 1) % 6, (NCHUNK - 1) % 3, (NCHUNK - 1) % 2)

    plsc.subcore_barrier()
    for z in range(NRCHUNK // NS + 1):
        c = z * NS + sid

        @pl.when(c < NRCHUNK)
        def _():
            pltpu.sync_copy(acc.at[pl.ds(c * K, K)],
                            num_hbm.at[cid, pl.ds(c * K, K)])
            pltpu.sync_copy(dacc.at[pl.ds(c * K, K)],
                            den_hbm.at[cid, pl.ds(c * K, K)])


_agg = pl.kernel(
    _agg_body,
    out_type=[
        jax.ShapeDtypeStruct((NC, N, D), jnp.float32),
        jax.ShapeDtypeStruct((NC, N), jnp.float32),
    ],
    mesh=plsc.VectorSubcoreMesh(core_axis_name="c", subcore_axis_name="s"),
    compiler_params=pltpu.CompilerParams(
        needs_layout_passes=False, use_tc_tiling_on_sc=False),
    scratch_types=[
        pltpu.VMEM((6, K), jnp.int32),
        pltpu.VMEM((6, K), jnp.int32),
        pltpu.VMEM((2, K), jnp.float32),
        pltpu.VMEM((2, K), jnp.float32),
        pltpu.VMEM((2, K), jnp.float32),
        pltpu.VMEM((K, D // 2), jnp.int32),
        pltpu.VMEM((K, D // 2), jnp.int32),
        pltpu.VMEM((K, D // 2), jnp.int32),
        pltpu.VMEM((K, D), jnp.float32),
        pltpu.VMEM((K, D), jnp.float32),
        pltpu.VMEM_SHARED((N, D), jnp.float32),
        pltpu.VMEM_SHARED((N,), jnp.float32),
        pltpu.SemaphoreType.DMA,
        pltpu.SemaphoreType.DMA,
        pltpu.SemaphoreType.DMA,
        pltpu.SemaphoreType.DMA,
        pltpu.SemaphoreType.DMA,
        pltpu.SemaphoreType.DMA,
        pltpu.SemaphoreType.DMA,
        pltpu.SemaphoreType.DMA,
        pltpu.SemaphoreType.DMA,
        pltpu.SemaphoreType.DMA,
        pltpu.SemaphoreType.DMA,
        pltpu.SemaphoreType.DMA,
        pltpu.SemaphoreType.DMA,
        pltpu.SemaphoreType.DMA,
        pltpu.SemaphoreType.DMA,
    ],
)


def _comb_body(p_ref, d_ref, o_ref):
    num = p_ref[0] + p_ref[1]
    den = d_ref[0] + d_ref[1]
    o_ref[...] = num / (den + 1e-8)


_comb = pl.pallas_call(
    _comb_body,
    out_shape=jax.ShapeDtypeStruct((N, D), jnp.float32),
)


def kernel(features, W, b, a, edge_index, nodes, ind):
    a2d = jnp.concatenate([a[:D], a[D:]], axis=1)
    packed, alphas = _prep(features, W, b.reshape(1, D), a2d)
    r3 = edge_index[0].reshape(NW, NCHUNK, K)
    c3 = edge_index[1].reshape(NW, NCHUNK, K)
    num, den = _agg(packed, alphas[:, 0], alphas[:, 1], r3, c3)
    return _comb(num, den.reshape(NC, N, 1))


# speedup vs baseline: 1.9692x; 1.7415x over previous
import jax
import jax.numpy as jnp
from jax import lax
from jax.experimental import pallas as pl
from jax.experimental.pallas import tpu as pltpu
from jax.experimental.pallas import tpu_sc as plsc

N = 10000
E = 320000
D = 128
SLOPE = 0.1

NC, NS, L = 2, 16, 16
NW = NC * NS
EPW = E // NW
K = 80
NCHUNK = EPW // K
NRCHUNK = N // K
UNROLL = 6
NLOOP = 120


def _rne16(u):
    return u + 0x7FFF + ((u >> 16) & 1)


def _prep_body(f_ref, w_ref, b_ref, a_ref, emb_ref, al_ref):
    emb = jnp.dot(f_ref[...], w_ref[...], preferred_element_type=jnp.float32)
    emb = emb + b_ref[...]
    al_ref[...] = jnp.dot(emb, a_ref[...], preferred_element_type=jnp.float32)
    u = lax.bitcast_convert_type(emb, jnp.int32)
    lo = (_rne16(u[:, :D // 2]) >> 16) & 0xFFFF
    hi = _rne16(u[:, D // 2:]) & -65536
    emb_ref[...] = lo | hi


_prep = pl.pallas_call(
    _prep_body,
    out_shape=[
        jax.ShapeDtypeStruct((N, D // 2), jnp.int32),
        jax.ShapeDtypeStruct((N, 2), jnp.float32),
    ],
)


def _agg_body(emb_hbm, ar_hbm, ac_hbm, r_hbm, c_hbm, num_hbm, den_hbm,
              rbuf, cbuf, abuf, bbuf, ebuf, rows0, rows1, rows2,
              stage0, stage1, acc, dacc,
              gsem0, gsem1, gsem2, ssem0, ssem1,
              isem0, isem1, isem2, isem3, isem4, isem5,
              asem0, asem1, esem0, esem1):
    cid = lax.axis_index("c")
    sid = lax.axis_index("s")
    wid = sid * NC + cid
    rows = (rows0, rows1, rows2)
    stage = (stage0, stage1)
    gsem = (gsem0, gsem1, gsem2)
    ssem = (ssem0, ssem1)
    isem = (isem0, isem1, isem2, isem3, isem4, isem5)
    asem = (asem0, asem1)
    esem = (esem0, esem1)

    z16 = jnp.zeros((L,), jnp.float32)

    def zrow(i, carry):
        for ci in range(D // L):
            stage0[i, pl.ds(ci * L, L)] = z16
        return carry

    lax.fori_loop(0, K, zrow, 0)
    for z in range(NRCHUNK // NS + 1):
        c = z * NS + sid

        @pl.when(c < NRCHUNK)
        def _():
            pltpu.sync_copy(stage0, acc.at[pl.ds(c * K, K)])
            pltpu.sync_copy(stage0.at[0, pl.ds(0, K)], dacc.at[pl.ds(c * K, K)])

    plsc.subcore_barrier()

    def idx_issue(j, q):
        pltpu.async_copy(r_hbm.at[wid, j], rbuf.at[q], isem[q])
        pltpu.async_copy(c_hbm.at[wid, j], cbuf.at[q], isem[q])

    def idx_wait(j, q):
        pltpu.make_async_copy(r_hbm.at[wid, j], rbuf.at[q], isem[q]).wait()
        pltpu.make_async_copy(c_hbm.at[wid, j], cbuf.at[q], isem[q]).wait()

    def ar_issue(j, q, a):
        pltpu.async_copy(ar_hbm.at[rbuf.at[q]], abuf.at[a], asem[a])
        pltpu.async_copy(ac_hbm.at[cbuf.at[q]], bbuf.at[a], asem[a])

    def ar_wait(j, q, a):
        pltpu.make_async_copy(ar_hbm.at[rbuf.at[q]], abuf.at[a], asem[a]).wait()
        pltpu.make_async_copy(ac_hbm.at[cbuf.at[q]], bbuf.at[a], asem[a]).wait()

    def gather(j, q, p):
        pltpu.async_copy(emb_hbm.at[cbuf.at[q]], rows[p], gsem[p])

    def gwait(j, q, p):
        pltpu.make_async_copy(emb_hbm.at[cbuf.at[q]], rows[p], gsem[p]).wait()

    def scatter(j, q, p, a):
        pltpu.async_copy(stage[a], acc.at[rbuf.at[q]], ssem[a], add=True)
        pltpu.async_copy(ebuf.at[a], dacc.at[rbuf.at[q]], esem[a], add=True)

    def swait(j, q, p, a):
        pltpu.make_async_copy(stage[a], acc.at[rbuf.at[q]], ssem[a]).wait()
        pltpu.make_async_copy(ebuf.at[a], dacc.at[rbuf.at[q]], esem[a]).wait()

    mhi = jnp.full((L,), -65536, jnp.int32)

    def scale(j, a, p):
        rp = rows[p]
        st = stage[a]

        NB = D // (2 * L)

        def grp(g, carry):
            s = abuf[a, pl.ds(g * L, L)] + bbuf[a, pl.ds(g * L, L)]
            e16 = jnp.exp(jnp.maximum(s, s * SLOPE))
            ebuf[a, pl.ds(g * L, L)] = e16
            for h in range(2):
                ws = []
                for i in range(8):
                    k = g * L + h * 8 + i
                    for c0 in range(NB):
                        ws.append(rp[k, pl.ds(c0 * L, L)])
                for i in range(8):
                    t = h * 8 + i
                    ek = e16[t]
                    k = g * L + t
                    for c0 in range(NB):
                        w = ws[i * NB + c0]
                        lo = plsc.bitcast(w << 16, jnp.float32) * ek
                        hi = plsc.bitcast(w & mhi, jnp.float32) * ek
                        st[k, pl.ds(c0 * L, L)] = lo
                        st[k, pl.ds(D // 2 + c0 * L, L)] = hi
            return carry

        lax.fori_loop(0, K // L, grp, 0)

    for jj in range(5):
        idx_issue(jj, jj)
    idx_wait(0, 0)
    ar_issue(0, 0, 0)
    gather(0, 0, 0)
    idx_wait(1, 1)
    gather(1, 1, 1)

    def phase(j, m, guard_swait, has_p1, has_p2, has_p5):
        p, q, a = m % 3, m % 6, m % 2
        if has_p1:
            ar_issue(j + 1, (q + 1) % 6, (a + 1) % 2)
        gwait(j, q, p)
        ar_wait(j, q, a)
        scale(j, a, p)
        scatter(j, q, p, a)

        def _wait_prev():
            swait(j - 1, (q + 5) % 6, (p + 2) % 3, (a + 1) % 2)

        if guard_swait:
            pl.when(j > 0)(_wait_prev)
        else:
            _wait_prev()
        if has_p2:
            idx_wait(j + 2, (q + 2) % 6)
            gather(j + 2, (q + 2) % 6, (p + 2) % 3)
        if has_p5:
            idx_issue(j + 5, (q + 5) % 6)

    def sixpack(t, carry):
        j0 = t * UNROLL
        for m in range(UNROLL):
            phase(j0 + m, m, m == 0, True, True, True)
        return carry

    lax.fori_loop(0, NLOOP // UNROLL, sixpack, 0)

    for j in range(NLOOP, NCHUNK):
        m = j % UNROLL
        phase(j, m, False, j + 1 < NCHUNK, j + 2 < NCHUNK, j + 5 < NCHUNK)
    swait(NCHUNK - 1, (NCHUNK - 1) % 6, (NCHUNK - 1) % 3, (NCHUNK - 1) % 2)

    plsc.subcore_barrier()
    for z in range(NRCHUNK // NS + 1):
        c = z * NS + sid

        @pl.when(c < NRCHUNK)
        def _():
            pltpu.sync_copy(acc.at[pl.ds(c * K, K)],
                            num_hbm.at[cid, pl.ds(c * K, K)])
            pltpu.sync_copy(dacc.at[pl.ds(c * K, K)],
                            den_hbm.at[cid, pl.ds(c * K, K)])


_agg = pl.kernel(
    _agg_body,
    out_type=[
        jax.ShapeDtypeStruct((NC, N, D), jnp.float32),
        jax.ShapeDtypeStruct((NC, N), jnp.float32),
    ],
    mesh=plsc.VectorSubcoreMesh(core_axis_name="c", subcore_axis_name="s"),
    compiler_params=pltpu.CompilerParams(
        needs_layout_passes=False, use_tc_tiling_on_sc=False),
    scratch_types=[
        pltpu.VMEM((6, K), jnp.int32),
        pltpu.VMEM((6, K), jnp.int32),
        pltpu.VMEM((2, K), jnp.float32),
        pltpu.VMEM((2, K), jnp.float32),
        pltpu.VMEM((2, K), jnp.float32),
        pltpu.VMEM((K, D // 2), jnp.int32),
        pltpu.VMEM((K, D // 2), jnp.int32),
        pltpu.VMEM((K, D // 2), jnp.int32),
        pltpu.VMEM((K, D), jnp.float32),
        pltpu.VMEM((K, D), jnp.float32),
        pltpu.VMEM_SHARED((N, D), jnp.float32),
        pltpu.VMEM_SHARED((N,), jnp.float32),
        pltpu.SemaphoreType.DMA,
        pltpu.SemaphoreType.DMA,
        pltpu.SemaphoreType.DMA,
        pltpu.SemaphoreType.DMA,
        pltpu.SemaphoreType.DMA,
        pltpu.SemaphoreType.DMA,
        pltpu.SemaphoreType.DMA,
        pltpu.SemaphoreType.DMA,
        pltpu.SemaphoreType.DMA,
        pltpu.SemaphoreType.DMA,
        pltpu.SemaphoreType.DMA,
        pltpu.SemaphoreType.DMA,
        pltpu.SemaphoreType.DMA,
        pltpu.SemaphoreType.DMA,
        pltpu.SemaphoreType.DMA,
    ],
)


def _comb_body(p_ref, d_ref, o_ref):
    num = p_ref[0] + p_ref[1]
    den = d_ref[0] + d_ref[1]
    o_ref[...] = num / (den + 1e-8)


_comb = pl.pallas_call(
    _comb_body,
    out_shape=jax.ShapeDtypeStruct((N, D), jnp.float32),
)


def kernel(features, W, b, a, edge_index, nodes, ind):
    a2d = jnp.concatenate([a[:D], a[D:]], axis=1)
    packed, alphas = _prep(features, W, b.reshape(1, D), a2d)
    r3 = edge_index[0].reshape(NW, NCHUNK, K)
    c3 = edge_index[1].reshape(NW, NCHUNK, K)
    num, den = _agg(packed, alphas[:, 0], alphas[:, 1], r3, c3)
    return _comb(num, den.reshape(NC, N, 1))
